# baseline (speedup 1.0000x reference)
import jax
import jax.numpy as jnp
from jax.experimental import pallas as pl
from jax.experimental.pallas import tpu as pltpu

FEAT_SIZE = 38.0
THRESHOLD = 0.5

_B = 16
_HW5 = 7220
_NC = 80
_CH = 722
_NB = _HW5 // _CH
_G = _B * _NB


def _body(cs_ref, conf_ref, box_ref, prior_ref, boxo_ref, probs_ref, idx_ref):
    cs = cs_ref[...]
    conf = conf_ref[...]
    scores = cs * conf
    m = jnp.max(scores, axis=-1)
    iota = jax.lax.broadcasted_iota(jnp.int32, scores.shape, 2).astype(jnp.float32)
    amf = jnp.min(jnp.where(scores == m[..., None], iota, 128.0), axis=-1)
    am = amf.astype(jnp.int32)
    mask = m > THRESHOLD

    box = box_ref[...]
    prior = prior_ref[...]
    xy = box[..., :2] + prior[..., :2]
    wh = box[..., 2:] * prior[..., 2:]
    mins = xy - wh / 2.0
    maxs = xy + wh / 2.0
    corners = jnp.concatenate([mins, maxs], axis=-1) / FEAT_SIZE
    boxo_ref[...] = jnp.where(mask[..., None], corners, 0.0)
    probs_ref[...] = jnp.where(mask, m, 0.0)[..., None]
    idx_ref[...] = am[..., None]


def kernel(box, box_confidence, class_score, prior):
    cs = class_score.reshape(_G, _CH, _NC)
    conf = box_confidence.reshape(_G, _CH, 1)
    boxr = box.reshape(_G, _CH, 4)
    priorr = prior.reshape(_NB, _CH, 4)

    boxo, probs, idx = pl.pallas_call(
        _body,
        grid=(_G,),
        in_specs=[
            pl.BlockSpec((1, _CH, _NC), lambda g: (g, 0, 0)),
            pl.BlockSpec((1, _CH, 1), lambda g: (g, 0, 0)),
            pl.BlockSpec((1, _CH, 4), lambda g: (g, 0, 0)),
            pl.BlockSpec((1, _CH, 4), lambda g: (g % _NB, 0, 0)),
        ],
        out_specs=[
            pl.BlockSpec((1, _CH, 4), lambda g: (g, 0, 0)),
            pl.BlockSpec((1, _CH, 1), lambda g: (g, 0, 0)),
            pl.BlockSpec((1, _CH, 1), lambda g: (g, 0, 0)),
        ],
        out_shape=[
            jax.ShapeDtypeStruct((_G, _CH, 4), jnp.float32),
            jax.ShapeDtypeStruct((_G, _CH, 1), jnp.float32),
            jax.ShapeDtypeStruct((_G, _CH, 1), jnp.int32),
        ],
        compiler_params=pltpu.CompilerParams(
            dimension_semantics=("parallel",),
        ),
    )(cs, conf, boxr, priorr)

    box_out = boxo.reshape(16, 1444, 5, 4)
    probs_out = probs.reshape(16, 1444, 5)
    idx_out = idx.reshape(16, 1444, 5)
    return box_out, probs_out, idx_out


# speedup vs baseline: 1.2069x; 1.2069x over previous
import jax
import jax.numpy as jnp
from jax.experimental import pallas as pl
from jax.experimental.pallas import tpu as pltpu

FEAT_SIZE = 38.0
THRESHOLD = 0.5

_B = 16
_HW = 1444
_A = 5
_NC = 80
_CH = 722
_NBH = _HW // _CH


def _body(cs_ref, conf_ref, box_ref, prior_ref, boxo_ref, probs_ref, idx_ref):
    cs = cs_ref[...]
    conf = conf_ref[...]
    scores = cs * conf
    m = jnp.max(scores, axis=-1, keepdims=True)
    iota = jax.lax.broadcasted_iota(jnp.int32, scores.shape, 3).astype(jnp.float32)
    amf = jnp.min(jnp.where(scores == m, iota, 128.0), axis=-1, keepdims=True)
    am = amf.astype(jnp.int32)
    mask = m > THRESHOLD

    box = box_ref[...]
    prior = prior_ref[...]
    xy = box[..., :2] + prior[..., :2]
    wh = box[..., 2:] * prior[..., 2:]
    mins = xy - wh / 2.0
    maxs = xy + wh / 2.0
    corners = jnp.concatenate([mins, maxs], axis=-1) / FEAT_SIZE
    boxo_ref[...] = jnp.where(mask, corners, 0.0)
    probs_ref[...] = jnp.where(mask, m, 0.0)
    idx_ref[...] = am


def kernel(box, box_confidence, class_score, prior):
    prior4 = prior[None]

    boxo, probs, idx = pl.pallas_call(
        _body,
        grid=(_B, _NBH),
        in_specs=[
            pl.BlockSpec((1, _CH, _A, _NC), lambda b, h: (b, h, 0, 0)),
            pl.BlockSpec((1, _CH, _A, 1), lambda b, h: (b, h, 0, 0)),
            pl.BlockSpec((1, _CH, _A, 4), lambda b, h: (b, h, 0, 0)),
            pl.BlockSpec((1, _CH, _A, 4), lambda b, h: (0, h, 0, 0)),
        ],
        out_specs=[
            pl.BlockSpec((1, _CH, _A, 4), lambda b, h: (b, h, 0, 0)),
            pl.BlockSpec((1, _CH, _A, 1), lambda b, h: (b, h, 0, 0)),
            pl.BlockSpec((1, _CH, _A, 1), lambda b, h: (b, h, 0, 0)),
        ],
        out_shape=[
            jax.ShapeDtypeStruct((_B, _HW, _A, 4), jnp.float32),
            jax.ShapeDtypeStruct((_B, _HW, _A, 1), jnp.float32),
            jax.ShapeDtypeStruct((_B, _HW, _A, 1), jnp.int32),
        ],
        compiler_params=pltpu.CompilerParams(
            dimension_semantics=("parallel", "parallel"),
        ),
    )(class_score, box_confidence, box, prior4)

    return boxo, probs[..., 0], idx[..., 0]


# speedup vs baseline: 20.7203x; 17.1677x over previous
import jax
import jax.numpy as jnp
from jax.experimental import pallas as pl
from jax.experimental.pallas import tpu as pltpu

FEAT_SIZE = 38.0
THRESHOLD = 0.5

_B = 16
_HW = 1444
_A = 5
_NC = 80
_HB = 1444
_NBH = _HW // _HB


def _body(cs_ref, conf_ref, box_ref, prior_ref, boxo_ref, probs_ref, idx_ref):
    cs = cs_ref[...]
    conf = conf_ref[...]
    scores = cs * conf[:, :, None, :]
    m = jnp.max(scores, axis=2)
    iota = jax.lax.broadcasted_iota(jnp.int32, scores.shape, 2).astype(jnp.float32)
    amf = jnp.min(jnp.where(scores == m[:, :, None, :], iota, 128.0), axis=2)
    am = amf.astype(jnp.int32)
    mask = m > THRESHOLD

    box = box_ref[...]
    prior = prior_ref[...]
    xy = box[:, :, :2, :] + prior[None, :, :2, :]
    wh = box[:, :, 2:, :] * prior[None, :, 2:, :]
    mins = xy - wh / 2.0
    maxs = xy + wh / 2.0
    corners = jnp.concatenate([mins, maxs], axis=2) / FEAT_SIZE
    boxo_ref[...] = jnp.where(mask[:, :, None, :], corners, 0.0)
    probs_ref[...] = jnp.where(mask, m, 0.0)
    idx_ref[...] = am


def kernel(box, box_confidence, class_score, prior):
    cs_t = jnp.transpose(class_score, (0, 2, 3, 1))
    conf_t = jnp.transpose(box_confidence[..., 0], (0, 2, 1))
    box_t = jnp.transpose(box, (0, 2, 3, 1))
    prior_t = jnp.transpose(prior, (1, 2, 0))

    boxo_t, probs_t, idx_t = pl.pallas_call(
        _body,
        grid=(_B, _NBH),
        in_specs=[
            pl.BlockSpec((1, _A, _NC, _HB), lambda b, h: (b, 0, 0, h)),
            pl.BlockSpec((1, _A, _HB), lambda b, h: (b, 0, h)),
            pl.BlockSpec((1, _A, 4, _HB), lambda b, h: (b, 0, 0, h)),
            pl.BlockSpec((_A, 4, _HB), lambda b, h: (0, 0, h)),
        ],
        out_specs=[
            pl.BlockSpec((1, _A, 4, _HB), lambda b, h: (b, 0, 0, h)),
            pl.BlockSpec((1, _A, _HB), lambda b, h: (b, 0, h)),
            pl.BlockSpec((1, _A, _HB), lambda b, h: (b, 0, h)),
        ],
        out_shape=[
            jax.ShapeDtypeStruct((_B, _A, 4, _HW), jnp.float32),
            jax.ShapeDtypeStruct((_B, _A, _HW), jnp.float32),
            jax.ShapeDtypeStruct((_B, _A, _HW), jnp.int32),
        ],
        compiler_params=pltpu.CompilerParams(
            dimension_semantics=("parallel", "parallel"),
        ),
    )(cs_t, conf_t, box_t, prior_t)

    box_out = jnp.transpose(boxo_t, (0, 3, 1, 2))
    probs_out = jnp.transpose(probs_t, (0, 2, 1))
    idx_out = jnp.transpose(idx_t, (0, 2, 1))
    return box_out, probs_out, idx_out
